# baseline (speedup 1.0000x reference)
import functools

import jax
import jax.numpy as jnp
from jax import lax
from jax.experimental import pallas as pl
from jax.experimental.pallas import tpu as pltpu
from jax.experimental.pallas import tpu_sc as plsc

B = 16384
S = 100
FID = 1000000

SPLITS = (8192, 8192)
OFFS = (0, 8192)

NW = 32

_sc_mesh = plsc.VectorSubcoreMesh(core_axis_name="c", subcore_axis_name="s")


def _make_gather(start, nrows):
    rpw = nrows // NW
    nchunk = -(-rpw // 256)
    while rpw % nchunk:
        nchunk += 1
    chunk = rpw // nchunk

    @functools.partial(
        pl.kernel,
        out_type=jax.ShapeDtypeStruct((nrows, S), jnp.int32),
        mesh=_sc_mesh,
        scratch_types=[
            pltpu.VMEM((rpw, S), jnp.int32),
            pltpu.VMEM((chunk, S), jnp.int32),
            pltpu.SemaphoreType.DMA,
        ],
        compiler_params=pltpu.CompilerParams(use_tc_tiling_on_sc=True),
    )
    def _sc_gather(idx_hbm, tab_hbm, out_hbm, idx_v, vals_v, sem):
        w = lax.axis_index("s") * 2 + lax.axis_index("c")
        base = w * rpw
        pltpu.sync_copy(idx_hbm.at[pl.ds(start + base, rpw)], idx_v)

        def chunk_body(ci, carry):
            row0 = ci * chunk

            def fire(j, c):
                pltpu.async_copy(
                    tab_hbm.at[idx_v.at[row0 + j]], vals_v.at[j], sem)
                return c

            lax.fori_loop(0, chunk, fire, 0)

            def drain(j, c):
                pltpu.make_async_copy(
                    tab_hbm.at[idx_v.at[row0 + j]], vals_v.at[j], sem
                ).wait()
                return c

            lax.fori_loop(0, chunk, drain, 0)
            pltpu.sync_copy(vals_v, out_hbm.at[pl.ds(base + row0, chunk)])
            return carry

        lax.fori_loop(0, nchunk, chunk_body, 0)

    return _sc_gather


_gathers = [_make_gather(o, n) for o, n in zip(OFFS, SPLITS)]


BM = 4096
N1 = 640


_DN_T = (((1,), (1,)), ((), ()))
_DN_M = (((1,), (0,)), ((), ()))


def _tc_tower(nb, x_ref, lab_ref, w1_ref, b1_ref, cc_ref, w2_ref, b2_ref,
              w3_ref, gb_ref, pred_ref, part_ref, acc_ref):
    i = pl.program_id(0)
    xi = x_ref[...]
    x_sp = lax.bitcast_convert_type(
        xi & jnp.int32(-65536), jnp.float32).astype(jnp.bfloat16)
    x_ct = lax.bitcast_convert_type(
        xi << 16, jnp.float32).astype(jnp.bfloat16)

    h0 = lax.dot_general(w1_ref[...], x_sp, _DN_T,
                         preferred_element_type=jnp.float32)
    bias_row = h0[512:513, :]
    cp = lax.dot_general(cc_ref[...], x_ct, _DN_T,
                         preferred_element_type=jnp.float32)
    cert_row = cp[0:1, :]

    h = jnp.maximum(h0[0:512, :] + b1_ref[...], 0.0).astype(jnp.bfloat16)
    h = lax.dot_general(w2_ref[...], h, _DN_M,
                        preferred_element_type=jnp.float32)
    h = jnp.maximum(h + b2_ref[...], 0.0).astype(jnp.bfloat16)
    nn = lax.dot_general(w3_ref[...], h, _DN_M,
                         preferred_element_type=jnp.float32)

    logits = bias_row + gb_ref[0] + nn[0:1, :]
    pred_ref[0] = jax.nn.sigmoid(logits)

    raw = jax.nn.sigmoid(cert_row) + 0.5
    lab = lab_ref[0]
    xent = (jnp.maximum(logits, 0.0) - logits * lab
            + jnp.log1p(jnp.exp(-jnp.abs(logits))))
    pa = jnp.sum(xent * raw)
    pc = jnp.sum(raw)

    @pl.when(i == 0)
    def _init():
        acc_ref[0] = pa
        acc_ref[1] = pc

    @pl.when(i > 0)
    def _accum():
        acc_ref[0] += pa
        acc_ref[1] += pc

    @pl.when(i == nb - 1)
    def _fin():
        part_ref[0] = acc_ref[0]
        part_ref[1] = acc_ref[1]


def _make_tower(nrows):
    nb = nrows // BM
    return pl.pallas_call(
        functools.partial(_tc_tower, nb),
        grid=(nb,),
        in_specs=[
            pl.BlockSpec((BM, S), lambda i: (i, 0)),
            pl.BlockSpec((1, 1, BM), lambda i: (i, 0, 0)),
            pl.BlockSpec((N1, S), lambda i: (0, 0)),
            pl.BlockSpec((512, 1), lambda i: (0, 0)),
            pl.BlockSpec((8, S), lambda i: (0, 0)),
            pl.BlockSpec((256, 512), lambda i: (0, 0)),
            pl.BlockSpec((256, 1), lambda i: (0, 0)),
            pl.BlockSpec((8, 256), lambda i: (0, 0)),
            pl.BlockSpec(memory_space=pltpu.SMEM),
        ],
        out_specs=[
            pl.BlockSpec((1, 1, BM), lambda i: (i, 0, 0)),
            pl.BlockSpec(memory_space=pltpu.SMEM),
        ],
        out_shape=[
            jax.ShapeDtypeStruct((nb, 1, BM), jnp.float32),
            jax.ShapeDtypeStruct((2,), jnp.float32),
        ],
        scratch_shapes=[pltpu.SMEM((2,), jnp.float32)],
    )


_towers = {n: _make_tower(n) for n in set(SPLITS)}


def kernel(slot_bias_fid_index, label, sparse_bias, certain_bias,
           global_bias, W1, b1, W2, b2, W3, b3):
    sb = lax.bitcast_convert_type(
        sparse_bias.astype(jnp.bfloat16), jnp.uint16).astype(jnp.uint32)
    cb = lax.bitcast_convert_type(
        certain_bias.astype(jnp.bfloat16), jnp.uint16).astype(jnp.uint32)
    tab = lax.bitcast_convert_type((sb << 16) | cb, jnp.int32)

    bf = jnp.bfloat16
    one = jnp.float32(1.0).astype(bf)
    w1a = jnp.zeros((N1, S), bf).at[:512].set(W1.T.astype(bf))
    w1a = w1a.at[512].set(one)
    cc = jnp.zeros((8, S), bf).at[0].set(one)
    w3c = jnp.zeros((8, 256), bf).at[0].set(W3[:, 0].astype(bf))
    gb = (global_bias[0] + b3[0]).reshape(1)
    w2b = W2.T.astype(bf)
    b1r = b1.reshape(512, 1)
    b2r = b2.reshape(256, 1)

    preds, pas, pcs = [], [], []
    for h, (off, nrows) in enumerate(zip(OFFS, SPLITS)):
        lab_h = lax.slice(label, (off,), (off + nrows,))
        x_h = _gathers[h](slot_bias_fid_index, tab)
        pred_h, part_h = _towers[nrows](
            x_h, lab_h.reshape(nrows // BM, 1, BM),
            w1a, b1r, cc, w2b, b2r, w3c, gb)
        preds.append(pred_h.reshape(nrows))
        pas.append(part_h[0])
        pcs.append(part_h[1])

    pa = sum(pas)
    pc = sum(pcs)
    loss = pa * jnp.float32(B) / pc
    return jnp.concatenate(preds), loss


# speedup vs baseline: 1.0192x; 1.0192x over previous
import functools

import jax
import jax.numpy as jnp
from jax import lax
from jax.experimental import pallas as pl
from jax.experimental.pallas import tpu as pltpu
from jax.experimental.pallas import tpu_sc as plsc

B = 16384
S = 100
FID = 1000000

SPLITS = (8192, 8192)
OFFS = (0, 8192)

NW = 32

_sc_mesh = plsc.VectorSubcoreMesh(core_axis_name="c", subcore_axis_name="s")


def _make_gather(start, nrows):
    rpw = nrows // NW
    nchunk = -(-rpw // 256)
    while rpw % nchunk:
        nchunk += 1
    chunk = rpw // nchunk

    @functools.partial(
        pl.kernel,
        out_type=jax.ShapeDtypeStruct((nrows, S), jnp.int32),
        mesh=_sc_mesh,
        scratch_types=[
            pltpu.VMEM((rpw, S), jnp.int32),
            pltpu.VMEM((chunk, S), jnp.int32),
            pltpu.SemaphoreType.DMA,
        ],
        compiler_params=pltpu.CompilerParams(use_tc_tiling_on_sc=True),
    )
    def _sc_gather(idx_hbm, tab_hbm, out_hbm, idx_v, vals_v, sem):
        w = lax.axis_index("s") * 2 + lax.axis_index("c")
        base = w * rpw
        pltpu.sync_copy(idx_hbm.at[pl.ds(start + base, rpw)], idx_v)

        def chunk_body(ci, carry):
            row0 = ci * chunk

            def fire(j, c):
                pltpu.async_copy(
                    tab_hbm.at[idx_v.at[row0 + j]], vals_v.at[j], sem)
                return c

            lax.fori_loop(0, chunk, fire, 0)

            def drain(j, c):
                pltpu.make_async_copy(
                    tab_hbm.at[idx_v.at[row0 + j]], vals_v.at[j], sem
                ).wait()
                return c

            lax.fori_loop(0, chunk, drain, 0)
            pltpu.sync_copy(vals_v, out_hbm.at[pl.ds(base + row0, chunk)])
            return carry

        lax.fori_loop(0, nchunk, chunk_body, 0)

    return _sc_gather


_gathers = [_make_gather(o, n) for o, n in zip(OFFS, SPLITS)]


BM = 4096
N1 = 640


_DN_T = (((1,), (1,)), ((), ()))
_DN_M = (((1,), (0,)), ((), ()))


def _tc_tower(nb, prev_ref, x_ref, lab_ref, w1_ref, b1_ref, cc_ref, w2_ref,
              b2_ref, w3_ref, gb_ref, pred_ref, part_ref, acc_ref):
    del prev_ref
    i = pl.program_id(0)
    xi = x_ref[...]
    x_sp = lax.bitcast_convert_type(
        xi & jnp.int32(-65536), jnp.float32).astype(jnp.bfloat16)
    x_ct = lax.bitcast_convert_type(
        xi << 16, jnp.float32).astype(jnp.bfloat16)

    h0 = lax.dot_general(w1_ref[...], x_sp, _DN_T,
                         preferred_element_type=jnp.float32)
    bias_row = h0[512:513, :]
    cp = lax.dot_general(cc_ref[...], x_ct, _DN_T,
                         preferred_element_type=jnp.float32)
    cert_row = cp[0:1, :]

    h = jnp.maximum(h0[0:512, :] + b1_ref[...], 0.0).astype(jnp.bfloat16)
    h = lax.dot_general(w2_ref[...], h, _DN_M,
                        preferred_element_type=jnp.float32)
    h = jnp.maximum(h + b2_ref[...], 0.0).astype(jnp.bfloat16)
    nn = lax.dot_general(w3_ref[...], h, _DN_M,
                         preferred_element_type=jnp.float32)

    logits = bias_row + gb_ref[0] + nn[0:1, :]
    pred_ref[0] = jax.nn.sigmoid(logits)

    raw = jax.nn.sigmoid(cert_row) + 0.5
    lab = lab_ref[0]
    xent = (jnp.maximum(logits, 0.0) - logits * lab
            + jnp.log1p(jnp.exp(-jnp.abs(logits))))
    pa = jnp.sum(xent * raw)
    pc = jnp.sum(raw)

    @pl.when(i == 0)
    def _init():
        acc_ref[0] = pa
        acc_ref[1] = pc

    @pl.when(i > 0)
    def _accum():
        acc_ref[0] += pa
        acc_ref[1] += pc

    @pl.when(i == nb - 1)
    def _fin():
        part_ref[0] = acc_ref[0]
        part_ref[1] = acc_ref[1]


def _make_tower(nrows, blk_off):
    nb = nrows // BM
    return pl.pallas_call(
        functools.partial(_tc_tower, nb),
        grid=(nb,),
        in_specs=[
            pl.BlockSpec(memory_space=pltpu.MemorySpace.HBM),
            pl.BlockSpec((BM, S), lambda i: (i, 0)),
            pl.BlockSpec((1, 1, BM), lambda i: (i, 0, 0)),
            pl.BlockSpec((N1, S), lambda i: (0, 0)),
            pl.BlockSpec((512, 1), lambda i: (0, 0)),
            pl.BlockSpec((8, S), lambda i: (0, 0)),
            pl.BlockSpec((256, 512), lambda i: (0, 0)),
            pl.BlockSpec((256, 1), lambda i: (0, 0)),
            pl.BlockSpec((8, 256), lambda i: (0, 0)),
            pl.BlockSpec(memory_space=pltpu.SMEM),
        ],
        out_specs=[
            pl.BlockSpec((1, 1, BM), lambda i: (i + blk_off, 0, 0)),
            pl.BlockSpec(memory_space=pltpu.SMEM),
        ],
        out_shape=[
            jax.ShapeDtypeStruct((B // BM, 1, BM), jnp.float32),
            jax.ShapeDtypeStruct((2,), jnp.float32),
        ],
        scratch_shapes=[pltpu.SMEM((2,), jnp.float32)],
        input_output_aliases={0: 0},
    )


_towers = [_make_tower(n, o // BM) for o, n in zip(OFFS, SPLITS)]


def kernel(slot_bias_fid_index, label, sparse_bias, certain_bias,
           global_bias, W1, b1, W2, b2, W3, b3):
    sb = lax.bitcast_convert_type(
        sparse_bias.astype(jnp.bfloat16), jnp.uint16).astype(jnp.uint32)
    cb = lax.bitcast_convert_type(
        certain_bias.astype(jnp.bfloat16), jnp.uint16).astype(jnp.uint32)
    tab = lax.bitcast_convert_type((sb << 16) | cb, jnp.int32)

    bf = jnp.bfloat16
    one = jnp.float32(1.0).astype(bf)
    w1a = jnp.zeros((N1, S), bf).at[:512].set(W1.T.astype(bf))
    w1a = w1a.at[512].set(one)
    cc = jnp.zeros((8, S), bf).at[0].set(one)
    w3c = jnp.zeros((8, 256), bf).at[0].set(W3[:, 0].astype(bf))
    gb = (global_bias[0] + b3[0]).reshape(1)
    w2b = W2.T.astype(bf)
    b1r = b1.reshape(512, 1)
    b2r = b2.reshape(256, 1)

    pred_acc = jnp.zeros((B // BM, 1, BM), jnp.float32)
    pas, pcs = [], []
    for h, (off, nrows) in enumerate(zip(OFFS, SPLITS)):
        lab_h = lax.slice(label, (off,), (off + nrows,))
        x_h = _gathers[h](slot_bias_fid_index, tab)
        pred_acc, part_h = _towers[h](
            pred_acc, x_h, lab_h.reshape(nrows // BM, 1, BM),
            w1a, b1r, cc, w2b, b2r, w3c, gb)
        pas.append(part_h[0])
        pcs.append(part_h[1])

    pa = sum(pas)
    pc = sum(pcs)
    loss = pa * jnp.float32(B) / pc
    return pred_acc.reshape(B), loss
